# baseline (speedup 1.0000x reference)
import functools
import math

import jax
import jax.numpy as jnp
from jax import lax
from jax.experimental import pallas as pl
from jax.experimental.pallas import tpu as pltpu
from jax.experimental.pallas import tpu_sc as plsc

D_MODEL = 64
SCALE = math.sqrt(D_MODEL)
NC = 2
NS = 16
NW = NC * NS
CH = 128


def _make_gather_kernel(steps):
    mesh = plsc.VectorSubcoreMesh(core_axis_name="c", subcore_axis_name="s")
    n_rows = NW * steps * CH

    @functools.partial(
        pl.kernel,
        mesh=mesh,
        out_type=jax.ShapeDtypeStruct((n_rows // 2, 2 * D_MODEL), jnp.float32),
        scratch_types=[
            pltpu.VMEM((steps, CH), jnp.int32),
            pltpu.VMEM((steps, CH), jnp.int32),
            pltpu.VMEM((2, CH, 2 * D_MODEL), jnp.float32),
            pltpu.VMEM((2, CH // 2, 2 * D_MODEL), jnp.float32),
            [pltpu.SemaphoreType.DMA] * 2,
            [pltpu.SemaphoreType.DMA] * 2,
        ],
        compiler_params=pltpu.CompilerParams(
            use_tc_tiling_on_sc=False, needs_layout_passes=False
        ),
    )
    def gather_kernel(idx2_hbm, par_hbm, t2_hbm, y_hbm,
                      idx2_v, par_v, rows_v, y_v, gs, ss):
        wid = lax.axis_index("s") * NC + lax.axis_index("c")
        pltpu.sync_copy(idx2_hbm.at[wid], idx2_v)
        pltpu.sync_copy(par_hbm.at[wid], par_v)
        out_base = wid * steps

        def gather_start(j, b):
            pltpu.async_copy(t2_hbm.at[idx2_v.at[j]], rows_v.at[b], gs[b])

        def gather_wait(j, b):
            pltpu.make_async_copy(
                t2_hbm.at[idx2_v.at[j]], rows_v.at[b], gs[b]
            ).wait()

        def scatter_start(j, b):
            row0 = (out_base + j) * (CH // 2)
            pltpu.async_copy(
                y_v.at[b], y_hbm.at[pl.ds(row0, CH // 2)], ss[b]
            )

        def scatter_wait(j, b):
            row0 = (out_base + j) * (CH // 2)
            pltpu.make_async_copy(
                y_v.at[b], y_hbm.at[pl.ds(row0, CH // 2)], ss[b]
            ).wait()

        def select_scale(j, b):
            @plsc.parallel_loop(0, CH // 16, 1, unroll=2)
            def _(g):
                parg = par_v[j, pl.ds(g * 16, 16)]
                r0 = g * 16
                for l in range(16):
                    p = parg[l]
                    for c in range(D_MODEL // 16):
                        v = rows_v[b, r0 + l, pl.ds(p + c * 16, 16)]
                        y_v[b, (r0 + l) // 2,
                            pl.ds((l % 2) * D_MODEL + c * 16, 16)] = v * SCALE

        def process(j, b, wait_prev_scatter, prefetch):
            gather_wait(j, b)
            if wait_prev_scatter:
                scatter_wait(j - 2, b)
            select_scale(j, b)
            if prefetch:
                gather_start(j + 2, b)
            scatter_start(j, b)

        gather_start(0, 0)
        gather_start(1, 1)
        process(0, 0, False, True)
        process(1, 1, False, True)

        @pl.loop(2, steps - 2, step=2)
        def _(j0):
            process(j0, 0, True, True)
            process(j0 + 1, 1, True, True)

        process(steps - 2, 0, True, False)
        process(steps - 1, 1, True, False)
        scatter_wait(steps - 2, 0)
        scatter_wait(steps - 1, 1)

    return gather_kernel


def _transpose_body(y_ref, out_ref):
    at = y_ref[0].T
    for cg in range(8):
        t0 = at[cg * 8:cg * 8 + 8, :]
        t1 = at[64 + cg * 8:64 + cg * 8 + 8, :]
        out_ref[0, cg, 0] = jnp.stack([t0, t1], axis=-1).reshape(8, CH)


def _make_transpose_kernel(n_s, n_b):
    n_bt = n_b // CH
    return pl.pallas_call(
        _transpose_body,
        grid=(n_s, n_bt),
        in_specs=[
            pl.BlockSpec((1, CH // 2, CH), lambda s, bt: (s * n_bt + bt, 0, 0))
        ],
        out_specs=pl.BlockSpec(
            (1, D_MODEL // 8, 1, 8, CH), lambda s, bt: (s, 0, bt, 0, 0)
        ),
        out_shape=jax.ShapeDtypeStruct(
            (n_s, D_MODEL // 8, n_bt, 8, CH), jnp.float32
        ),
    )


def kernel(x, table):
    n_b, n_s = x.shape
    v, d = table.shape
    assert d == D_MODEL and v % 2 == 0 and n_b % CH == 0
    assert (n_s * n_b) % (NW * CH) == 0
    steps = (n_s * n_b) // (NW * CH)
    t2 = table.reshape(v // 2, 2 * D_MODEL)
    xt = x.T.reshape(NW, steps, CH)
    idx2 = xt >> 1
    par = (xt & 1) * D_MODEL
    y = _make_gather_kernel(steps)(idx2, par, t2)
    out5 = _make_transpose_kernel(n_s, n_b)(
        y.reshape(n_s * (n_b // CH), CH // 2, CH)
    )
    out = out5.transpose(2, 4, 0, 1, 3).reshape(n_b, n_s, D_MODEL)
    return out


# speedup vs baseline: 1.0097x; 1.0097x over previous
import functools
import math

import jax
import jax.numpy as jnp
from jax import lax
from jax.experimental import pallas as pl
from jax.experimental.pallas import tpu as pltpu
from jax.experimental.pallas import tpu_sc as plsc

D_MODEL = 64
SCALE = math.sqrt(D_MODEL)
NC = 2
NS = 16
NW = NC * NS
CH = 128


def _make_gather_kernel(steps):
    mesh = plsc.VectorSubcoreMesh(core_axis_name="c", subcore_axis_name="s")
    n_rows = NW * steps * CH

    @functools.partial(
        pl.kernel,
        mesh=mesh,
        out_type=jax.ShapeDtypeStruct((n_rows // 2, 2 * D_MODEL), jnp.float32),
        scratch_types=[
            pltpu.VMEM((steps, CH), jnp.int32),
            pltpu.VMEM((2, CH, 2 * D_MODEL), jnp.float32),
            pltpu.VMEM((2, CH // 2, 2 * D_MODEL), jnp.float32),
            [pltpu.SemaphoreType.DMA] * 2,
            [pltpu.SemaphoreType.DMA] * 2,
        ],
        compiler_params=pltpu.CompilerParams(
            use_tc_tiling_on_sc=False, needs_layout_passes=False
        ),
    )
    def gather_kernel(idx_hbm, t2_hbm, y_hbm, idx_v, rows_v, y_v, gs, ss):
        wid = lax.axis_index("s") * NC + lax.axis_index("c")
        pltpu.sync_copy(idx_hbm.at[wid], idx_v)
        out_base = wid * steps

        def gather_start(j, b):
            pltpu.async_copy(t2_hbm.at[idx_v.at[j]], rows_v.at[b], gs[b])

        def gather_wait(j, b):
            pltpu.make_async_copy(
                t2_hbm.at[idx_v.at[j]], rows_v.at[b], gs[b]
            ).wait()

        def scatter_start(j, b):
            row0 = (out_base + j) * (CH // 2)
            pltpu.async_copy(
                y_v.at[b], y_hbm.at[pl.ds(row0, CH // 2)], ss[b]
            )

        def scatter_wait(j, b):
            row0 = (out_base + j) * (CH // 2)
            pltpu.make_async_copy(
                y_v.at[b], y_hbm.at[pl.ds(row0, CH // 2)], ss[b]
            ).wait()

        def select_scale(j, b):
            @plsc.parallel_loop(0, CH, 1, unroll=4)
            def _(r):
                for c in range(D_MODEL // 16):
                    v = rows_v[b, r, pl.ds(c * 16, 16)]
                    y_v[b, r // 2,
                        pl.ds((r % 2) * D_MODEL + c * 16, 16)] = v * SCALE

        def process(j, b, wait_prev_scatter, prefetch):
            gather_wait(j, b)
            if wait_prev_scatter:
                scatter_wait(j - 2, b)
            select_scale(j, b)
            if prefetch:
                gather_start(j + 2, b)
            scatter_start(j, b)

        gather_start(0, 0)
        gather_start(1, 1)
        process(0, 0, False, True)
        process(1, 1, False, True)

        @pl.loop(2, steps - 2, step=2)
        def _(j0):
            process(j0, 0, True, True)
            process(j0 + 1, 1, True, True)

        process(steps - 2, 0, True, False)
        process(steps - 1, 1, True, False)
        scatter_wait(steps - 2, 0)
        scatter_wait(steps - 1, 1)

    return gather_kernel


def _transpose_body(y_ref, out_ref):
    at = y_ref[0].T
    for cg in range(8):
        t0 = at[cg * 8:cg * 8 + 8, :]
        t1 = at[64 + cg * 8:64 + cg * 8 + 8, :]
        out_ref[0, cg, 0] = jnp.stack([t0, t1], axis=-1).reshape(8, CH)


def _make_transpose_kernel(n_s, n_b):
    n_bt = n_b // CH
    return pl.pallas_call(
        _transpose_body,
        grid=(n_s, n_bt),
        in_specs=[
            pl.BlockSpec((1, CH // 2, CH), lambda s, bt: (s * n_bt + bt, 0, 0))
        ],
        out_specs=pl.BlockSpec(
            (1, D_MODEL // 8, 1, 8, CH), lambda s, bt: (s, 0, bt, 0, 0)
        ),
        out_shape=jax.ShapeDtypeStruct(
            (n_s, D_MODEL // 8, n_bt, 8, CH), jnp.float32
        ),
    )


def kernel(x, table):
    n_b, n_s = x.shape
    v, d = table.shape
    assert d == D_MODEL and n_b % CH == 0
    assert (n_s * n_b) % (NW * CH) == 0
    steps = (n_s * n_b) // (NW * CH)
    t2 = jnp.pad(table, ((0, 0), (0, D_MODEL)))
    idx = x.T.reshape(NW, steps, CH)
    y = _make_gather_kernel(steps)(idx, t2)
    out5 = _make_transpose_kernel(n_s, n_b)(
        y.reshape(n_s * (n_b // CH), CH // 2, CH)
    )
    out = out5.transpose(2, 4, 0, 1, 3).reshape(n_b, n_s, D_MODEL)
    return out


# speedup vs baseline: 1.3754x; 1.3622x over previous
import functools
import math

import jax
import jax.numpy as jnp
from jax import lax
from jax.experimental import pallas as pl
from jax.experimental.pallas import tpu as pltpu
from jax.experimental.pallas import tpu_sc as plsc

D_MODEL = 64
SCALE = math.sqrt(D_MODEL)
NC = 2
NS = 16
NW = NC * NS
CH = 128


def _make_gather_kernel(steps):
    mesh = plsc.VectorSubcoreMesh(core_axis_name="c", subcore_axis_name="s")
    n_rows = NW * steps * CH

    @functools.partial(
        pl.kernel,
        mesh=mesh,
        out_type=jax.ShapeDtypeStruct((n_rows // 2, 2 * D_MODEL), jnp.float32),
        scratch_types=[
            pltpu.VMEM((steps, CH), jnp.int32),
            pltpu.VMEM((2, CH, 2 * D_MODEL), jnp.float32),
            pltpu.VMEM((2, CH // 2, 2 * D_MODEL), jnp.float32),
            [pltpu.SemaphoreType.DMA] * 2,
            [pltpu.SemaphoreType.DMA] * 2,
        ],
        compiler_params=pltpu.CompilerParams(
            use_tc_tiling_on_sc=False, needs_layout_passes=False
        ),
    )
    def gather_kernel(idx_hbm, t2_hbm, y_hbm, idx_v, rows_v, y_v, gs, ss):
        wid = lax.axis_index("s") * NC + lax.axis_index("c")
        pltpu.sync_copy(idx_hbm.at[wid], idx_v)
        out_base = wid * steps

        def gather_start(j, b):
            pltpu.async_copy(t2_hbm.at[idx_v.at[j]], rows_v.at[b], gs[b])

        def gather_wait(j, b):
            pltpu.make_async_copy(
                t2_hbm.at[idx_v.at[j]], rows_v.at[b], gs[b]
            ).wait()

        def scatter_start(j, b):
            row0 = (out_base + j) * (CH // 2)
            pltpu.async_copy(
                y_v.at[b], y_hbm.at[pl.ds(row0, CH // 2)], ss[b]
            )

        def scatter_wait(j, b):
            row0 = (out_base + j) * (CH // 2)
            pltpu.make_async_copy(
                y_v.at[b], y_hbm.at[pl.ds(row0, CH // 2)], ss[b]
            ).wait()

        def select_scale(j, b):
            @plsc.parallel_loop(0, CH, 1, unroll=4)
            def _(r):
                for c in range(D_MODEL // 16):
                    v = rows_v[b, r, pl.ds(c * 16, 16)]
                    y_v[b, r // 2,
                        pl.ds((r % 2) * D_MODEL + c * 16, 16)] = v * SCALE

        def process(j, b, wait_prev_scatter, prefetch):
            gather_wait(j, b)
            if wait_prev_scatter:
                scatter_wait(j - 2, b)
            select_scale(j, b)
            if prefetch:
                gather_start(j + 2, b)
            scatter_start(j, b)

        gather_start(0, 0)
        gather_start(1, 1)
        process(0, 0, False, True)
        process(1, 1, False, True)

        @pl.loop(2, steps - 2, step=2)
        def _(j0):
            process(j0, 0, True, True)
            process(j0 + 1, 1, True, True)

        process(steps - 2, 0, True, False)
        process(steps - 1, 1, True, False)
        scatter_wait(steps - 2, 0)
        scatter_wait(steps - 1, 1)

    return gather_kernel


def _transpose_body(y_ref, out_ref):
    z = y_ref[0].reshape(64, 2, 8, 8)
    out_ref[0] = z.transpose(2, 3, 0, 1).reshape(8, 1, 8, CH)


def _make_transpose_kernel(n_s, n_b):
    n_bt = n_b // CH
    return pl.pallas_call(
        _transpose_body,
        grid=(n_s, n_bt),
        in_specs=[
            pl.BlockSpec((1, CH // 2, CH), lambda s, bt: (s * n_bt + bt, 0, 0))
        ],
        out_specs=pl.BlockSpec(
            (1, D_MODEL // 8, 1, 8, CH), lambda s, bt: (s, 0, bt, 0, 0)
        ),
        out_shape=jax.ShapeDtypeStruct(
            (n_s, D_MODEL // 8, n_bt, 8, CH), jnp.float32
        ),
    )


def kernel(x, table):
    n_b, n_s = x.shape
    v, d = table.shape
    assert d == D_MODEL and n_b % CH == 0
    assert (n_s * n_b) % (NW * CH) == 0
    steps = (n_s * n_b) // (NW * CH)
    t2 = jnp.pad(table, ((0, 0), (0, D_MODEL)))
    idx = x.T.reshape(NW, steps, CH)
    y = _make_gather_kernel(steps)(idx, t2)
    out5 = _make_transpose_kernel(n_s, n_b)(
        y.reshape(n_s * (n_b // CH), CH // 2, CH)
    )
    out = out5.transpose(2, 4, 0, 1, 3).reshape(n_b, n_s, D_MODEL)
    return out


# speedup vs baseline: 12.1954x; 8.8668x over previous
import functools
import math

import jax
import jax.numpy as jnp
from jax import lax
from jax.experimental import pallas as pl
from jax.experimental.pallas import tpu as pltpu
from jax.experimental.pallas import tpu_sc as plsc

D_MODEL = 64
SCALE = math.sqrt(D_MODEL)
NC = 2
NS = 16
NW = NC * NS
CH = 128


def _make_kernel(steps):
    mesh = plsc.VectorSubcoreMesh(core_axis_name="c", subcore_axis_name="s")
    n_rows = NW * steps * CH

    @functools.partial(
        pl.kernel,
        mesh=mesh,
        out_type=jax.ShapeDtypeStruct((n_rows, D_MODEL), jnp.float32),
        scratch_types=[
            pltpu.VMEM((steps, CH), jnp.int32),
            pltpu.VMEM((2, CH, D_MODEL), jnp.float32),
            [pltpu.SemaphoreType.DMA] * 2,
            [pltpu.SemaphoreType.DMA] * 2,
        ],
        compiler_params=pltpu.CompilerParams(use_tc_tiling_on_sc=False),
    )
    def emb_kernel(idx_hbm, table_hbm, out_hbm, idx_v, rows_v, gs, ss):
        wid = lax.axis_index("s") * NC + lax.axis_index("c")
        pltpu.sync_copy(idx_hbm.at[wid], idx_v)
        out_base = wid * steps

        def gather_start(j, b):
            pltpu.async_copy(table_hbm.at[idx_v.at[j]], rows_v.at[b], gs[b])

        def gather_wait(j, b):
            pltpu.make_async_copy(
                table_hbm.at[idx_v.at[j]], rows_v.at[b], gs[b]
            ).wait()

        def scatter_start(j, b):
            row0 = (out_base + j) * CH
            pltpu.async_copy(rows_v.at[b], out_hbm.at[pl.ds(row0, CH)], ss[b])

        def scatter_wait(j, b):
            row0 = (out_base + j) * CH
            pltpu.make_async_copy(
                rows_v.at[b], out_hbm.at[pl.ds(row0, CH)], ss[b]
            ).wait()

        def scale(b):
            @plsc.parallel_loop(0, CH, 1, unroll=4)
            def _(r):
                for c in range(D_MODEL // 16):
                    sl = pl.ds(c * 16, 16)
                    rows_v[b, r, sl] = rows_v[b, r, sl] * SCALE

        def process(j, b, wait_prev_scatter, prefetch):
            gather_wait(j, b)
            if wait_prev_scatter:
                scatter_wait(j - 2, b)
            scale(b)
            if prefetch:
                gather_start(j + 2, b)
            scatter_start(j, b)

        gather_start(0, 0)
        gather_start(1, 1)
        process(0, 0, False, True)
        process(1, 1, False, True)

        @pl.loop(2, steps - 2, step=2)
        def _(j0):
            process(j0, 0, True, True)
            process(j0 + 1, 1, True, True)

        process(steps - 2, 0, True, False)
        process(steps - 1, 1, True, False)
        scatter_wait(steps - 2, 0)
        scatter_wait(steps - 1, 1)

    return emb_kernel


def kernel(x, table):
    n_b, n_s = x.shape
    v, d = table.shape
    assert d == D_MODEL and (n_b * n_s) % (NW * CH) == 0
    steps = (n_b * n_s) // (NW * CH)
    idx = x.reshape(NW, steps, CH)
    out = _make_kernel(steps)(idx, table)
    return out.reshape(n_b, n_s, D_MODEL)
